# baseline (speedup 1.0000x reference)
import jax
import jax.numpy as jnp
from jax.experimental import pallas as pl
from jax.experimental.pallas import tpu as pltpu

T_STEPS = 12
K = 3
N = 1024
C_IN = 32
HID = 64
B = 16
GOUT = 4 * HID


def _step_kernel(xT_ref, LT_ref, Wx_ref, Wh_ref, bx_ref, bh_ref, bg_ref,
                 wc_ref, one_ref, h0_ref, c0_ref, out_ref, h_s, c_s):
    t = pl.program_id(0)
    o1 = one_ref[0, 0]

    @pl.when(t == 0)
    def _init():
        h_s[...] = h0_ref[...]
        c_s[...] = c0_ref[...]

    LT = LT_ref[...]

    X0m = xT_ref[0].reshape(B * C_IN, N)
    X1m = jnp.dot(X0m, LT, preferred_element_type=jnp.float32)
    X2m = 2.0 * jnp.dot(X1m, LT, preferred_element_type=jnp.float32) - X0m
    X0 = xT_ref[0]
    X1 = X1m.reshape(B, C_IN, N)
    X2 = X2m.reshape(B, C_IN, N)

    Hm = h_s[...].reshape(B * HID, N)
    H1h = []
    H2h = []
    for g in range(2):
        h0m = Hm[g * 512:(g + 1) * 512]
        h1m = jnp.dot(h0m, LT, preferred_element_type=jnp.float32)
        h2m = 2.0 * jnp.dot(h1m, LT, preferred_element_type=jnp.float32) - h0m
        H1h.append(h1m)
        H2h.append(h2m)
    H0 = h_s[...]
    H1 = jnp.concatenate(H1h, axis=0).reshape(B, HID, N)
    H2 = jnp.concatenate(H2h, axis=0).reshape(B, HID, N)

    bxT = bx_ref[0]
    bhT = bh_ref[0]
    wc_i = wc_ref[0, 0]
    wc_f = wc_ref[0, 1]
    wc_o = wc_ref[0, 2]
    bg_i = bg_ref[0, :HID]
    bg_f = bg_ref[0, HID:2 * HID]
    bg_c = bg_ref[0, 2 * HID:3 * HID]
    bg_o = bg_ref[0, 3 * HID:]

    for b in range(B):
        xw = (jnp.dot(Wx_ref[0, 0], X0[b], preferred_element_type=jnp.float32)
              + jnp.dot(Wx_ref[0, 1], X1[b], preferred_element_type=jnp.float32))
        xw = (xw + jnp.dot(Wx_ref[0, 2], X2[b], preferred_element_type=jnp.float32)) + bxT
        hw = (jnp.dot(Wh_ref[0, 0], H0[b], preferred_element_type=jnp.float32)
              + jnp.dot(Wh_ref[0, 1], H1[b], preferred_element_type=jnp.float32))
        hw = (hw + jnp.dot(Wh_ref[0, 2], H2[b], preferred_element_type=jnp.float32)) + bhT
        pre = o1 * xw + o1 * hw

        c_b = c_s[b]
        Ig = jax.nn.sigmoid((pre[:HID] + wc_i * c_b) + bg_i)
        Fg = jax.nn.sigmoid((pre[HID:2 * HID] + wc_f * c_b) + bg_f)
        Tc = jnp.tanh(pre[2 * HID:3 * HID] + bg_c)
        c_new = Fg * c_b + Ig * Tc
        Og = jax.nn.sigmoid((pre[3 * HID:] + wc_o * c_new) + bg_o)
        h_new = Og * jnp.tanh(c_new)

        c_s[b] = c_new
        h_s[b] = h_new
        out_ref[b, 0] = h_new.T


def kernel(X, L, H, C, Wx, bx, Wh, bh, wc, bg):
    LT = L.T
    XT = jnp.transpose(X, (1, 0, 3, 2))
    WxS = jnp.transpose(Wx, (0, 2, 1, 4, 3)).reshape(T_STEPS, K, GOUT, C_IN)
    WhS = jnp.transpose(Wh, (0, 2, 1, 4, 3)).reshape(T_STEPS, K, GOUT, HID)
    bxS = bx.reshape(T_STEPS, GOUT, 1)
    bhS = bh.reshape(T_STEPS, GOUT, 1)
    bgS = bg[:, :, 0].reshape(T_STEPS, GOUT, 1)
    wcr = wc[:, :, 0].reshape(T_STEPS, 3, HID, 1)
    h0T = jnp.transpose(H[0], (0, 2, 1))
    c0T = jnp.transpose(C[0], (0, 2, 1))
    one = jnp.ones((8, 128), jnp.float32)

    out = pl.pallas_call(
        _step_kernel,
        grid=(T_STEPS,),
        in_specs=[
            pl.BlockSpec((1, B, C_IN, N), lambda t: (t, 0, 0, 0)),
            pl.BlockSpec((N, N), lambda t: (0, 0)),
            pl.BlockSpec((1, K, GOUT, C_IN), lambda t: (t, 0, 0, 0)),
            pl.BlockSpec((1, K, GOUT, HID), lambda t: (t, 0, 0, 0)),
            pl.BlockSpec((1, GOUT, 1), lambda t: (t, 0, 0)),
            pl.BlockSpec((1, GOUT, 1), lambda t: (t, 0, 0)),
            pl.BlockSpec((1, GOUT, 1), lambda t: (t, 0, 0)),
            pl.BlockSpec((1, 3, HID, 1), lambda t: (t, 0, 0, 0)),
            pl.BlockSpec((8, 128), lambda t: (0, 0)),
            pl.BlockSpec((B, HID, N), lambda t: (0, 0, 0)),
            pl.BlockSpec((B, HID, N), lambda t: (0, 0, 0)),
        ],
        out_specs=pl.BlockSpec((B, 1, N, HID), lambda t: (0, t, 0, 0)),
        out_shape=jax.ShapeDtypeStruct((B, T_STEPS, N, HID), jnp.float32),
        scratch_shapes=[
            pltpu.VMEM((B, HID, N), jnp.float32),
            pltpu.VMEM((B, HID, N), jnp.float32),
        ],
        compiler_params=pltpu.CompilerParams(
            dimension_semantics=("arbitrary",),
        ),
    )(XT, LT, WxS, WhS, bxS, bhS, bgS, wcr, one, h0T, c0T)
    return out


# speedup vs baseline: 1.2028x; 1.2028x over previous
import jax
import jax.numpy as jnp
from jax.experimental import pallas as pl
from jax.experimental.pallas import tpu as pltpu

T_STEPS = 12
K = 3
N = 1024
C_IN = 32
HID = 64
B = 16
GOUT = 4 * HID


def _step_kernel(xT_ref, LT_ref, Wx_ref, Wh_ref, bx_ref, bh_ref, bg_ref,
                 wc_ref, one_ref, h0_ref, c0_ref, out_ref, h_s, c_s):
    t = pl.program_id(0)
    o1 = one_ref[0, 0]

    @pl.when(t == 0)
    def _init():
        h_s[...] = h0_ref[...]
        c_s[...] = c0_ref[...]

    LT = LT_ref[...]

    X0m = xT_ref[0].reshape(B * C_IN, N)
    X1m = jnp.dot(X0m, LT, preferred_element_type=jnp.float32)
    X2m = 2.0 * jnp.dot(X1m, LT, preferred_element_type=jnp.float32) - X0m
    X0 = xT_ref[0]
    X1 = X1m.reshape(B, C_IN, N)
    X2 = X2m.reshape(B, C_IN, N)

    Hm = h_s[...].reshape(B * HID, N)
    H1h = []
    H2h = []
    for g in range(2):
        h0m = Hm[g * 512:(g + 1) * 512]
        h1m = jnp.dot(h0m, LT, preferred_element_type=jnp.float32)
        h2m = 2.0 * jnp.dot(h1m, LT, preferred_element_type=jnp.float32) - h0m
        H1h.append(h1m)
        H2h.append(h2m)
    H0 = h_s[...]
    H1 = jnp.concatenate(H1h, axis=0).reshape(B, HID, N)
    H2 = jnp.concatenate(H2h, axis=0).reshape(B, HID, N)

    bxT = bx_ref[0]
    bhT = bh_ref[0]
    wc_i = wc_ref[0, 0]
    wc_f = wc_ref[0, 1]
    wc_o = wc_ref[0, 2]
    bg_i = bg_ref[0, :HID]
    bg_f = bg_ref[0, HID:2 * HID]
    bg_c = bg_ref[0, 2 * HID:3 * HID]
    bg_o = bg_ref[0, 3 * HID:]

    for b in range(B):
        xw = (jnp.dot(Wx_ref[0, 0], X0[b], preferred_element_type=jnp.float32)
              + jnp.dot(Wx_ref[0, 1], X1[b], preferred_element_type=jnp.float32))
        xw = (xw + jnp.dot(Wx_ref[0, 2], X2[b], preferred_element_type=jnp.float32)) + bxT
        hw = (jnp.dot(Wh_ref[0, 0], H0[b], preferred_element_type=jnp.float32)
              + jnp.dot(Wh_ref[0, 1], H1[b], preferred_element_type=jnp.float32))
        hw = (hw + jnp.dot(Wh_ref[0, 2], H2[b], preferred_element_type=jnp.float32)) + bhT
        pre = o1 * xw + o1 * hw

        c_b = c_s[b]
        Ig = jax.nn.sigmoid((pre[:HID] + wc_i * c_b) + bg_i)
        Fg = jax.nn.sigmoid((pre[HID:2 * HID] + wc_f * c_b) + bg_f)
        Tc = jnp.tanh(pre[2 * HID:3 * HID] + bg_c)
        c_new = Fg * c_b + Ig * Tc
        Og = jax.nn.sigmoid((pre[3 * HID:] + wc_o * c_new) + bg_o)
        h_new = Og * jnp.tanh(c_new)

        c_s[b] = c_new
        h_s[b] = h_new
        out_ref[b, 0] = h_new


def kernel(X, L, H, C, Wx, bx, Wh, bh, wc, bg):
    LT = L.T
    XT = jnp.transpose(X, (1, 0, 3, 2))
    WxS = jnp.transpose(Wx, (0, 2, 1, 4, 3)).reshape(T_STEPS, K, GOUT, C_IN)
    WhS = jnp.transpose(Wh, (0, 2, 1, 4, 3)).reshape(T_STEPS, K, GOUT, HID)
    bxS = bx.reshape(T_STEPS, GOUT, 1)
    bhS = bh.reshape(T_STEPS, GOUT, 1)
    bgS = bg[:, :, 0].reshape(T_STEPS, GOUT, 1)
    wcr = wc[:, :, 0].reshape(T_STEPS, 3, HID, 1)
    h0T = jnp.transpose(H[0], (0, 2, 1))
    c0T = jnp.transpose(C[0], (0, 2, 1))
    one = jnp.ones((8, 128), jnp.float32)

    out = pl.pallas_call(
        _step_kernel,
        grid=(T_STEPS,),
        in_specs=[
            pl.BlockSpec((1, B, C_IN, N), lambda t: (t, 0, 0, 0)),
            pl.BlockSpec((N, N), lambda t: (0, 0)),
            pl.BlockSpec((1, K, GOUT, C_IN), lambda t: (t, 0, 0, 0)),
            pl.BlockSpec((1, K, GOUT, HID), lambda t: (t, 0, 0, 0)),
            pl.BlockSpec((1, GOUT, 1), lambda t: (t, 0, 0)),
            pl.BlockSpec((1, GOUT, 1), lambda t: (t, 0, 0)),
            pl.BlockSpec((1, GOUT, 1), lambda t: (t, 0, 0)),
            pl.BlockSpec((1, 3, HID, 1), lambda t: (t, 0, 0, 0)),
            pl.BlockSpec((8, 128), lambda t: (0, 0)),
            pl.BlockSpec((B, HID, N), lambda t: (0, 0, 0)),
            pl.BlockSpec((B, HID, N), lambda t: (0, 0, 0)),
        ],
        out_specs=pl.BlockSpec((B, 1, HID, N), lambda t: (0, t, 0, 0)),
        out_shape=jax.ShapeDtypeStruct((B, T_STEPS, HID, N), jnp.float32),
        scratch_shapes=[
            pltpu.VMEM((B, HID, N), jnp.float32),
            pltpu.VMEM((B, HID, N), jnp.float32),
        ],
        compiler_params=pltpu.CompilerParams(
            dimension_semantics=("arbitrary",),
        ),
    )(XT, LT, WxS, WhS, bxS, bhS, bgS, wcr, one, h0T, c0T)
    return jnp.transpose(out, (0, 1, 3, 2))
